# baseline (speedup 1.0000x reference)
import jax
import jax.numpy as jnp
from jax.experimental import pallas as pl
from jax.experimental.pallas import tpu as pltpu

NE = 4
B = 256
NOBJ = 16
N = 17
AG = 8
DYN = 12
STAT = 4
EMB = 32
HID = 64
ACT = 8

BB = 64


def _gnn_kernel(agent_ref, dyn_ref, stat_ref, act_ref,
                W_ea_ref, b_ea_ref, W_eo_ref, b_eo_ref,
                W_e1_ref, b_e1_ref, g_e_ref, be_e_ref, W_e2_ref, b_e2_ref,
                W_n1_ref, b_n1_ref, g_n_ref, be_n_ref, W_n2_ref, b_n2_ref,
                W_oa_ref, b_oa_ref, W_od_ref, b_od_ref,
                agent_out_ref, obj_out_ref):
    f32 = jnp.float32

    def mm(x, w):
        return jnp.dot(x, w, preferred_element_type=f32)

    def ln_relu(h, g, bb):
        m = jnp.mean(h, axis=-1, keepdims=True)
        v = jnp.mean(jnp.square(h - m), axis=-1, keepdims=True)
        return jnp.maximum((h - m) * jax.lax.rsqrt(v + 1e-5) * g + bb, 0.0)

    J = jnp.full((HID, HID), 1.0 / HID, dtype=f32)

    def ln_relu2(h2d, g, bb):
        m = mm(h2d, J)
        q = mm(h2d * h2d, J)
        inv = jax.lax.rsqrt(q - m * m + 1e-5)
        return jnp.maximum((h2d - m) * inv * g + bb, 0.0)

    ag = agent_ref[0]
    act = act_ref[0]
    obj_inT = jnp.concatenate([dyn_ref[0], stat_ref[0]], axis=-1).transpose(1, 0, 2)

    agent_emb = mm(ag, W_ea_ref[0]) + b_ea_ref[0]
    obj_embT = mm(obj_inT.reshape(NOBJ * BB, DYN + STAT), W_eo_ref[0]) + b_eo_ref[0]
    nfT = jnp.concatenate([agent_emb, obj_embT], axis=0)

    W_e1 = W_e1_ref[0]
    Pd = mm(nfT, W_e1[0:EMB]).reshape(N, BB, HID)
    Qs = mm(nfT, W_e1[EMB:2 * EMB]).reshape(N, BB, HID)
    R = mm(act, W_e1[2 * EMB:]) + b_e1_ref[0]
    Pd = Pd + R[None, :, :]

    mQ = jnp.mean(Qs, axis=-1, keepdims=True)
    mP = jnp.mean(Pd, axis=-1, keepdims=True)
    Qs_c = Qs - mQ
    Pd_c = Pd - mP
    vQ = jnp.mean(Qs_c * Qs_c, axis=-1)
    vP = jnp.mean(Pd_c * Pd_c, axis=-1)
    crossb = jax.lax.dot_general(
        Qs_c.transpose(1, 0, 2), Pd_c.transpose(1, 0, 2),
        (((2,), (2,)), ((0,), (0,))),
        precision=jax.lax.Precision.HIGHEST)
    crossT = crossb.transpose(1, 2, 0)
    var = vQ[:, None, :] + vP[None, :, :] + crossT * (2.0 / HID)
    inv = jax.lax.rsqrt(var + 1e-5)

    g_e = g_e_ref[0]
    be_e = be_e_ref[0]
    h_c = Qs_c[:, None, :, :] + Pd_c[None, :, :, :]
    t = jnp.maximum(h_c * inv[:, :, :, None] * g_e + be_e, 0.0)

    eyeN = jnp.where(jax.lax.broadcasted_iota(jnp.int32, (N, N), 0) ==
                     jax.lax.broadcasted_iota(jnp.int32, (N, N), 1), 1.0, 0.0)
    inv_d = jnp.sum(inv * eyeN[:, :, None], axis=1)
    t_diag = jnp.maximum((Qs_c + Pd_c) * inv_d[:, :, None] * g_e + be_e, 0.0)
    s_full = jnp.sum(t.reshape(N, N * BB, HID), axis=0)
    s = (s_full - t_diag.reshape(N * BB, HID)) * (1.0 / (N - 1))

    W_n1 = W_n1_ref[0]
    Wn_g = W_n1[EMB + ACT:]
    W_eg = mm(W_e2_ref[0], Wn_g)
    c_eg = mm(b_e2_ref[0], Wn_g)
    U = mm(nfT, W_n1[0:EMB])
    V = mm(act, W_n1[EMB:EMB + ACT]) + c_eg + b_n1_ref[0]
    G = mm(s, W_eg)
    h2 = (U + G).reshape(N, BB, HID) + V[None, :, :]
    t2 = ln_relu2(h2.reshape(N * BB, HID), g_n_ref[0], be_n_ref[0])
    t2 = t2.reshape(N, BB, HID)

    W_n2 = W_n2_ref[0]
    b_n2 = b_n2_ref[0]
    W_a = mm(W_n2, W_oa_ref[0])
    c_a = mm(b_n2, W_oa_ref[0]) + b_oa_ref[0]
    W_d = mm(W_n2, W_od_ref[0])
    c_d = mm(b_n2, W_od_ref[0]) + b_od_ref[0]
    agent_out_ref[0] = mm(t2[0], W_a) + c_a
    obj = t2[1:].reshape(NOBJ * BB, HID)
    obj_out_ref[0] = (mm(obj, W_d) + c_d).reshape(NOBJ, BB, DYN).transpose(1, 0, 2)


def kernel(agent_state, object_dyn_state, object_stat_state, action,
           W_ea, b_ea, W_eo, b_eo,
           W_e1, b_e1, g_e, be_e, W_e2, b_e2,
           W_n1, b_n1, g_n, be_n, W_n2, b_n2,
           W_oa, b_oa, W_od, b_od):
    grid = (NE, B // BB)

    b_ea, b_eo, b_e1, g_e, be_e, b_e2, b_n1, g_n, be_n, b_n2, b_oa, b_od = (
        x[:, None, :] for x in
        (b_ea, b_eo, b_e1, g_e, be_e, b_e2, b_n1, g_n, be_n, b_n2, b_oa, b_od))

    def eb(*blk):
        return pl.BlockSpec(blk, lambda e, b: (e, b) + (0,) * (len(blk) - 2))

    def ew(*blk):
        return pl.BlockSpec(blk, lambda e, b: (e,) + (0,) * (len(blk) - 1))

    in_specs = [
        eb(1, BB, AG),
        eb(1, BB, NOBJ, DYN),
        eb(1, BB, NOBJ, STAT),
        eb(1, BB, ACT),
        ew(1, AG, EMB), ew(1, 1, EMB),
        ew(1, DYN + STAT, EMB), ew(1, 1, EMB),
        ew(1, 2 * EMB + ACT, HID), ew(1, 1, HID),
        ew(1, 1, HID), ew(1, 1, HID),
        ew(1, HID, HID), ew(1, 1, HID),
        ew(1, EMB + HID + ACT, HID), ew(1, 1, HID),
        ew(1, 1, HID), ew(1, 1, HID),
        ew(1, HID, EMB), ew(1, 1, EMB),
        ew(1, EMB, AG), ew(1, 1, AG),
        ew(1, EMB, DYN), ew(1, 1, DYN),
    ]
    out_specs = (
        eb(1, BB, AG),
        eb(1, BB, NOBJ, DYN),
    )
    out_shapes = (
        jax.ShapeDtypeStruct((NE, B, AG), jnp.float32),
        jax.ShapeDtypeStruct((NE, B, NOBJ, DYN), jnp.float32),
    )
    return pl.pallas_call(
        _gnn_kernel,
        grid=grid,
        in_specs=in_specs,
        out_specs=out_specs,
        out_shape=out_shapes,
        compiler_params=pltpu.CompilerParams(
            dimension_semantics=("parallel", "parallel"),
        ),
    )(agent_state, object_dyn_state, object_stat_state, action,
      W_ea, b_ea, W_eo, b_eo,
      W_e1, b_e1, g_e, be_e, W_e2, b_e2,
      W_n1, b_n1, g_n, be_n, W_n2, b_n2,
      W_oa, b_oa, W_od, b_od)


# speedup vs baseline: 2.6379x; 2.6379x over previous
import jax
import jax.numpy as jnp
from jax.experimental import pallas as pl
from jax.experimental.pallas import tpu as pltpu

NE = 4
B = 256
NOBJ = 16
N = 17
AG = 8
DYN = 12
STAT = 4
EMB = 32
HID = 64
ACT = 8

BB = 64
HB = BB // 2


def _gnn_kernel(agent_ref, obj_ref, act_ref,
                W_ea_ref, b_ea_ref, W_eo_ref, b_eo_ref,
                W_e1_ref, b_e1_ref, g_e_ref, be_e_ref, W_e2_ref, b_e2_ref,
                W_n1_ref, b_n1_ref, g_n_ref, be_n_ref, W_n2_ref, b_n2_ref,
                W_oa_ref, b_oa_ref, W_od_ref, b_od_ref,
                agent_out_ref, obj_out_ref):
    f32 = jnp.float32

    def mm(x, w):
        return jnp.dot(x, w, preferred_element_type=f32)

    def bd(w):
        z = jnp.zeros_like(w)
        return jnp.concatenate([jnp.concatenate([w, z], axis=1),
                                jnp.concatenate([z, w], axis=1)], axis=0)

    def dup(v):
        return jnp.concatenate([v, v], axis=-1)

    J2 = bd(jnp.full((HID, HID), 1.0 / HID, dtype=f32))

    ag = agent_ref[0]
    act = act_ref[0]
    obj_in = obj_ref[0].reshape(NOBJ * HB, 2 * (DYN + STAT))

    agent_emb = mm(ag, bd(W_ea_ref[0])) + dup(b_ea_ref[0])
    obj_emb = mm(obj_in, bd(W_eo_ref[0])) + dup(b_eo_ref[0])
    nfTp = jnp.concatenate([agent_emb, obj_emb], axis=0)

    W_e1 = W_e1_ref[0]
    Pd = mm(nfTp, bd(W_e1[0:EMB])).reshape(N, HB, 2 * HID)
    Qs = mm(nfTp, bd(W_e1[EMB:2 * EMB])).reshape(N, HB, 2 * HID)
    R = mm(act, bd(W_e1[2 * EMB:])) + dup(b_e1_ref[0])
    Pd = Pd + R[None, :, :]

    mQ = mm(Qs.reshape(N * HB, 2 * HID), J2)
    mP = mm(Pd.reshape(N * HB, 2 * HID), J2)
    Qs_c = (Qs.reshape(N * HB, 2 * HID) - mQ).reshape(N, HB, 2 * HID)
    Pd_c = (Pd.reshape(N * HB, 2 * HID) - mP).reshape(N, HB, 2 * HID)

    g2 = dup(g_e_ref[0])
    be2 = dup(be_e_ref[0])

    h_c = Qs_c[:, None, :, :] + Pd_c[None, :, :, :]
    h2d = h_c.reshape(N * N * HB, 2 * HID)
    var = mm(h2d * h2d, J2)
    t = jnp.maximum(h2d * jax.lax.rsqrt(var + 1e-5) * g2 + be2, 0.0)

    hd = (Qs_c + Pd_c).reshape(N * HB, 2 * HID)
    var_d = mm(hd * hd, J2)
    t_diag = jnp.maximum(hd * jax.lax.rsqrt(var_d + 1e-5) * g2 + be2, 0.0)
    s_full = jnp.sum(t.reshape(N, N * HB, 2 * HID), axis=0)
    s = (s_full - t_diag) * (1.0 / (N - 1))

    W_n1 = W_n1_ref[0]
    Wn_g = W_n1[EMB + ACT:]
    W_eg = mm(W_e2_ref[0], Wn_g)
    U = mm(nfTp, bd(W_n1[0:EMB]))
    V = mm(act, bd(W_n1[EMB:EMB + ACT]))
    V = V + dup(mm(b_e2_ref[0], Wn_g) + b_n1_ref[0])
    G = mm(s, bd(W_eg))
    h2 = ((U + G).reshape(N, HB, 2 * HID) + V[None, :, :]).reshape(N * HB, 2 * HID)
    m2 = mm(h2, J2)
    q2 = mm(h2 * h2, J2)
    inv2 = jax.lax.rsqrt(q2 - m2 * m2 + 1e-5)
    t2 = jnp.maximum((h2 - m2) * inv2 * dup(g_n_ref[0]) + dup(be_n_ref[0]), 0.0)
    t2 = t2.reshape(N, HB, 2 * HID)

    W_n2 = W_n2_ref[0]
    b_n2 = b_n2_ref[0]
    W_a = mm(W_n2, W_oa_ref[0])
    c_a = mm(b_n2, W_oa_ref[0]) + b_oa_ref[0]
    W_d = mm(W_n2, W_od_ref[0])
    c_d = mm(b_n2, W_od_ref[0]) + b_od_ref[0]
    agent_out_ref[0] = mm(t2[0], bd(W_a)) + dup(c_a)
    obj = t2[1:].reshape(NOBJ * HB, 2 * HID)
    o_out = mm(obj, bd(W_d)) + dup(c_d)
    obj_out_ref[0] = o_out.reshape(NOBJ, HB, 2 * DYN)


def kernel(agent_state, object_dyn_state, object_stat_state, action,
           W_ea, b_ea, W_eo, b_eo,
           W_e1, b_e1, g_e, be_e, W_e2, b_e2,
           W_n1, b_n1, g_n, be_n, W_n2, b_n2,
           W_oa, b_oa, W_od, b_od):
    grid = (NE, B // BB)

    ag_p = agent_state.reshape(NE, B // 2, 2 * AG)
    act_p = action.reshape(NE, B // 2, 2 * ACT)
    obj_cat = jnp.concatenate([object_dyn_state, object_stat_state], axis=-1)
    obj_p = obj_cat.transpose(0, 2, 1, 3).reshape(NE, NOBJ, B // 2, 2 * (DYN + STAT))

    b_ea, b_eo, b_e1, g_e, be_e, b_e2, b_n1, g_n, be_n, b_n2, b_oa, b_od = (
        x[:, None, :] for x in
        (b_ea, b_eo, b_e1, g_e, be_e, b_e2, b_n1, g_n, be_n, b_n2, b_oa, b_od))

    def eb(*blk):
        return pl.BlockSpec(blk, lambda e, b: (e, b) + (0,) * (len(blk) - 2))

    def ebo(*blk):
        return pl.BlockSpec(blk, lambda e, b: (e, 0, b, 0))

    def ew(*blk):
        return pl.BlockSpec(blk, lambda e, b: (e,) + (0,) * (len(blk) - 1))

    in_specs = [
        eb(1, HB, 2 * AG),
        ebo(1, NOBJ, HB, 2 * (DYN + STAT)),
        eb(1, HB, 2 * ACT),
        ew(1, AG, EMB), ew(1, 1, EMB),
        ew(1, DYN + STAT, EMB), ew(1, 1, EMB),
        ew(1, 2 * EMB + ACT, HID), ew(1, 1, HID),
        ew(1, 1, HID), ew(1, 1, HID),
        ew(1, HID, HID), ew(1, 1, HID),
        ew(1, EMB + HID + ACT, HID), ew(1, 1, HID),
        ew(1, 1, HID), ew(1, 1, HID),
        ew(1, HID, EMB), ew(1, 1, EMB),
        ew(1, EMB, AG), ew(1, 1, AG),
        ew(1, EMB, DYN), ew(1, 1, DYN),
    ]
    out_specs = (
        eb(1, HB, 2 * AG),
        ebo(1, NOBJ, HB, 2 * DYN),
    )
    out_shapes = (
        jax.ShapeDtypeStruct((NE, B // 2, 2 * AG), jnp.float32),
        jax.ShapeDtypeStruct((NE, NOBJ, B // 2, 2 * DYN), jnp.float32),
    )
    a_out, o_out = pl.pallas_call(
        _gnn_kernel,
        grid=grid,
        in_specs=in_specs,
        out_specs=out_specs,
        out_shape=out_shapes,
        compiler_params=pltpu.CompilerParams(
            dimension_semantics=("parallel", "parallel"),
        ),
    )(ag_p, obj_p, act_p,
      W_ea, b_ea, W_eo, b_eo,
      W_e1, b_e1, g_e, be_e, W_e2, b_e2,
      W_n1, b_n1, g_n, be_n, W_n2, b_n2,
      W_oa, b_oa, W_od, b_od)

    agent_out = a_out.reshape(NE, B, AG)
    obj_out = o_out.reshape(NE, NOBJ, B, DYN).transpose(0, 2, 1, 3)
    return (agent_out, obj_out)


# speedup vs baseline: 2.9510x; 1.1187x over previous
import jax
import jax.numpy as jnp
from jax.experimental import pallas as pl
from jax.experimental.pallas import tpu as pltpu

NE = 4
B = 256
NOBJ = 16
N = 17
AG = 8
DYN = 12
STAT = 4
EMB = 32
HID = 64
ACT = 8

BB = 128
HB = BB // 2


def _gnn_kernel(agent_ref, obj_ref, act_ref,
                W_ea_ref, b_ea_ref, W_eo_ref, b_eo_ref,
                W_e1_ref, b_e1_ref, g_e_ref, be_e_ref, W_e2_ref, b_e2_ref,
                W_n1_ref, b_n1_ref, g_n_ref, be_n_ref, W_n2_ref, b_n2_ref,
                W_oa_ref, b_oa_ref, W_od_ref, b_od_ref,
                agent_out_ref, obj_out_ref):
    f32 = jnp.float32

    def mm(x, w):
        return jnp.dot(x, w, preferred_element_type=f32)

    def bd(w):
        z = jnp.zeros_like(w)
        return jnp.concatenate([jnp.concatenate([w, z], axis=1),
                                jnp.concatenate([z, w], axis=1)], axis=0)

    def dup(v):
        return jnp.concatenate([v, v], axis=-1)

    J2 = bd(jnp.full((HID, HID), 1.0 / HID, dtype=f32))

    ag = agent_ref[0]
    act = act_ref[0]
    obj_in = obj_ref[0].reshape(NOBJ * HB, 2 * (DYN + STAT))

    agent_emb = mm(ag, bd(W_ea_ref[0])) + dup(b_ea_ref[0])
    obj_emb = mm(obj_in, bd(W_eo_ref[0])) + dup(b_eo_ref[0])
    nfTp = jnp.concatenate([agent_emb, obj_emb], axis=0)

    W_e1 = W_e1_ref[0]
    Pd = mm(nfTp, bd(W_e1[0:EMB])).reshape(N, HB, 2 * HID)
    Qs = mm(nfTp, bd(W_e1[EMB:2 * EMB])).reshape(N, HB, 2 * HID)
    R = mm(act, bd(W_e1[2 * EMB:])) + dup(b_e1_ref[0])
    Pd = Pd + R[None, :, :]

    mQ = mm(Qs.reshape(N * HB, 2 * HID), J2)
    mP = mm(Pd.reshape(N * HB, 2 * HID), J2)
    Qs_c = (Qs.reshape(N * HB, 2 * HID) - mQ).reshape(N, HB, 2 * HID)
    Pd_c = (Pd.reshape(N * HB, 2 * HID) - mP).reshape(N, HB, 2 * HID)

    g2 = dup(g_e_ref[0])
    be2 = dup(be_e_ref[0])

    h_c = Qs_c[:, None, :, :] + Pd_c[None, :, :, :]
    h2d = h_c.reshape(N * N * HB, 2 * HID)
    var = mm(h2d * h2d, J2)
    t = jnp.maximum(h2d * jax.lax.rsqrt(var + 1e-5) * g2 + be2, 0.0)

    hd = (Qs_c + Pd_c).reshape(N * HB, 2 * HID)
    var_d = mm(hd * hd, J2)
    t_diag = jnp.maximum(hd * jax.lax.rsqrt(var_d + 1e-5) * g2 + be2, 0.0)
    s_full = jnp.sum(t.reshape(N, N * HB, 2 * HID), axis=0)
    s = (s_full - t_diag) * (1.0 / (N - 1))

    W_n1 = W_n1_ref[0]
    Wn_g = W_n1[EMB + ACT:]
    W_eg = mm(W_e2_ref[0], Wn_g)
    U = mm(nfTp, bd(W_n1[0:EMB]))
    V = mm(act, bd(W_n1[EMB:EMB + ACT]))
    V = V + dup(mm(b_e2_ref[0], Wn_g) + b_n1_ref[0])
    G = mm(s, bd(W_eg))
    h2 = ((U + G).reshape(N, HB, 2 * HID) + V[None, :, :]).reshape(N * HB, 2 * HID)
    m2 = mm(h2, J2)
    q2 = mm(h2 * h2, J2)
    inv2 = jax.lax.rsqrt(q2 - m2 * m2 + 1e-5)
    t2 = jnp.maximum((h2 - m2) * inv2 * dup(g_n_ref[0]) + dup(be_n_ref[0]), 0.0)
    t2 = t2.reshape(N, HB, 2 * HID)

    W_n2 = W_n2_ref[0]
    b_n2 = b_n2_ref[0]
    W_a = mm(W_n2, W_oa_ref[0])
    c_a = mm(b_n2, W_oa_ref[0]) + b_oa_ref[0]
    W_d = mm(W_n2, W_od_ref[0])
    c_d = mm(b_n2, W_od_ref[0]) + b_od_ref[0]
    agent_out_ref[0] = mm(t2[0], bd(W_a)) + dup(c_a)
    obj = t2[1:].reshape(NOBJ * HB, 2 * HID)
    o_out = mm(obj, bd(W_d)) + dup(c_d)
    obj_out_ref[0] = o_out.reshape(NOBJ, HB, 2 * DYN)


def kernel(agent_state, object_dyn_state, object_stat_state, action,
           W_ea, b_ea, W_eo, b_eo,
           W_e1, b_e1, g_e, be_e, W_e2, b_e2,
           W_n1, b_n1, g_n, be_n, W_n2, b_n2,
           W_oa, b_oa, W_od, b_od):
    grid = (NE, B // BB)

    ag_p = agent_state.reshape(NE, B // 2, 2 * AG)
    act_p = action.reshape(NE, B // 2, 2 * ACT)
    obj_cat = jnp.concatenate([object_dyn_state, object_stat_state], axis=-1)
    obj_p = obj_cat.transpose(0, 2, 1, 3).reshape(NE, NOBJ, B // 2, 2 * (DYN + STAT))

    b_ea, b_eo, b_e1, g_e, be_e, b_e2, b_n1, g_n, be_n, b_n2, b_oa, b_od = (
        x[:, None, :] for x in
        (b_ea, b_eo, b_e1, g_e, be_e, b_e2, b_n1, g_n, be_n, b_n2, b_oa, b_od))

    def eb(*blk):
        return pl.BlockSpec(blk, lambda e, b: (e, b) + (0,) * (len(blk) - 2))

    def ebo(*blk):
        return pl.BlockSpec(blk, lambda e, b: (e, 0, b, 0))

    def ew(*blk):
        return pl.BlockSpec(blk, lambda e, b: (e,) + (0,) * (len(blk) - 1))

    in_specs = [
        eb(1, HB, 2 * AG),
        ebo(1, NOBJ, HB, 2 * (DYN + STAT)),
        eb(1, HB, 2 * ACT),
        ew(1, AG, EMB), ew(1, 1, EMB),
        ew(1, DYN + STAT, EMB), ew(1, 1, EMB),
        ew(1, 2 * EMB + ACT, HID), ew(1, 1, HID),
        ew(1, 1, HID), ew(1, 1, HID),
        ew(1, HID, HID), ew(1, 1, HID),
        ew(1, EMB + HID + ACT, HID), ew(1, 1, HID),
        ew(1, 1, HID), ew(1, 1, HID),
        ew(1, HID, EMB), ew(1, 1, EMB),
        ew(1, EMB, AG), ew(1, 1, AG),
        ew(1, EMB, DYN), ew(1, 1, DYN),
    ]
    out_specs = (
        eb(1, HB, 2 * AG),
        ebo(1, NOBJ, HB, 2 * DYN),
    )
    out_shapes = (
        jax.ShapeDtypeStruct((NE, B // 2, 2 * AG), jnp.float32),
        jax.ShapeDtypeStruct((NE, NOBJ, B // 2, 2 * DYN), jnp.float32),
    )
    a_out, o_out = pl.pallas_call(
        _gnn_kernel,
        grid=grid,
        in_specs=in_specs,
        out_specs=out_specs,
        out_shape=out_shapes,
        compiler_params=pltpu.CompilerParams(
            dimension_semantics=("parallel", "parallel"),
        ),
    )(ag_p, obj_p, act_p,
      W_ea, b_ea, W_eo, b_eo,
      W_e1, b_e1, g_e, be_e, W_e2, b_e2,
      W_n1, b_n1, g_n, be_n, W_n2, b_n2,
      W_oa, b_oa, W_od, b_od)

    agent_out = a_out.reshape(NE, B, AG)
    obj_out = o_out.reshape(NE, NOBJ, B, DYN).transpose(0, 2, 1, 3)
    return (agent_out, obj_out)


# speedup vs baseline: 3.0497x; 1.0335x over previous
import jax
import jax.numpy as jnp
from jax.experimental import pallas as pl
from jax.experimental.pallas import tpu as pltpu

NE = 4
B = 256
NOBJ = 16
N = 17
AG = 8
DYN = 12
STAT = 4
EMB = 32
HID = 64
ACT = 8

BB = 256
HB = BB // 2


def _gnn_kernel(agent_ref, obj_ref, act_ref,
                W_ea_ref, b_ea_ref, W_eo_ref, b_eo_ref,
                W_e1_ref, b_e1_ref, g_e_ref, be_e_ref, W_e2_ref, b_e2_ref,
                W_n1_ref, b_n1_ref, g_n_ref, be_n_ref, W_n2_ref, b_n2_ref,
                W_oa_ref, b_oa_ref, W_od_ref, b_od_ref,
                agent_out_ref, obj_out_ref):
    f32 = jnp.float32

    def mm(x, w):
        return jnp.dot(x, w, preferred_element_type=f32)

    def bd(w):
        z = jnp.zeros_like(w)
        return jnp.concatenate([jnp.concatenate([w, z], axis=1),
                                jnp.concatenate([z, w], axis=1)], axis=0)

    def dup(v):
        return jnp.concatenate([v, v], axis=-1)

    J2 = bd(jnp.full((HID, HID), 1.0 / HID, dtype=f32))

    ag = agent_ref[0]
    act = act_ref[0]
    obj_in = obj_ref[0].reshape(NOBJ * HB, 2 * (DYN + STAT))

    agent_emb = mm(ag, bd(W_ea_ref[0])) + dup(b_ea_ref[0])
    obj_emb = mm(obj_in, bd(W_eo_ref[0])) + dup(b_eo_ref[0])
    nfTp = jnp.concatenate([agent_emb, obj_emb], axis=0)

    W_e1 = W_e1_ref[0]
    Pd = mm(nfTp, bd(W_e1[0:EMB])).reshape(N, HB, 2 * HID)
    Qs = mm(nfTp, bd(W_e1[EMB:2 * EMB])).reshape(N, HB, 2 * HID)
    R = mm(act, bd(W_e1[2 * EMB:])) + dup(b_e1_ref[0])
    Pd = Pd + R[None, :, :]

    mQ = mm(Qs.reshape(N * HB, 2 * HID), J2)
    mP = mm(Pd.reshape(N * HB, 2 * HID), J2)
    Qs_c = (Qs.reshape(N * HB, 2 * HID) - mQ).reshape(N, HB, 2 * HID)
    Pd_c = (Pd.reshape(N * HB, 2 * HID) - mP).reshape(N, HB, 2 * HID)

    g2 = dup(g_e_ref[0])
    be2 = dup(be_e_ref[0])

    h_c = Qs_c[:, None, :, :] + Pd_c[None, :, :, :]
    h2d = h_c.reshape(N * N * HB, 2 * HID)
    var = mm(h2d * h2d, J2)
    t = jnp.maximum(h2d * jax.lax.rsqrt(var + 1e-5) * g2 + be2, 0.0)

    hd = (Qs_c + Pd_c).reshape(N * HB, 2 * HID)
    var_d = mm(hd * hd, J2)
    t_diag = jnp.maximum(hd * jax.lax.rsqrt(var_d + 1e-5) * g2 + be2, 0.0)
    s_full = jnp.sum(t.reshape(N, N * HB, 2 * HID), axis=0)
    s = (s_full - t_diag) * (1.0 / (N - 1))

    W_n1 = W_n1_ref[0]
    Wn_g = W_n1[EMB + ACT:]
    W_eg = mm(W_e2_ref[0], Wn_g)
    U = mm(nfTp, bd(W_n1[0:EMB]))
    V = mm(act, bd(W_n1[EMB:EMB + ACT]))
    V = V + dup(mm(b_e2_ref[0], Wn_g) + b_n1_ref[0])
    G = mm(s, bd(W_eg))
    h2 = ((U + G).reshape(N, HB, 2 * HID) + V[None, :, :]).reshape(N * HB, 2 * HID)
    m2 = mm(h2, J2)
    q2 = mm(h2 * h2, J2)
    inv2 = jax.lax.rsqrt(q2 - m2 * m2 + 1e-5)
    t2 = jnp.maximum((h2 - m2) * inv2 * dup(g_n_ref[0]) + dup(be_n_ref[0]), 0.0)
    t2 = t2.reshape(N, HB, 2 * HID)

    W_n2 = W_n2_ref[0]
    b_n2 = b_n2_ref[0]
    W_a = mm(W_n2, W_oa_ref[0])
    c_a = mm(b_n2, W_oa_ref[0]) + b_oa_ref[0]
    W_d = mm(W_n2, W_od_ref[0])
    c_d = mm(b_n2, W_od_ref[0]) + b_od_ref[0]
    agent_out_ref[0] = mm(t2[0], bd(W_a)) + dup(c_a)
    obj = t2[1:].reshape(NOBJ * HB, 2 * HID)
    o_out = mm(obj, bd(W_d)) + dup(c_d)
    obj_out_ref[0] = o_out.reshape(NOBJ, HB, 2 * DYN)


def kernel(agent_state, object_dyn_state, object_stat_state, action,
           W_ea, b_ea, W_eo, b_eo,
           W_e1, b_e1, g_e, be_e, W_e2, b_e2,
           W_n1, b_n1, g_n, be_n, W_n2, b_n2,
           W_oa, b_oa, W_od, b_od):
    grid = (NE, B // BB)

    ag_p = agent_state.reshape(NE, B // 2, 2 * AG)
    act_p = action.reshape(NE, B // 2, 2 * ACT)
    obj_cat = jnp.concatenate([object_dyn_state, object_stat_state], axis=-1)
    obj_p = obj_cat.transpose(0, 2, 1, 3).reshape(NE, NOBJ, B // 2, 2 * (DYN + STAT))

    b_ea, b_eo, b_e1, g_e, be_e, b_e2, b_n1, g_n, be_n, b_n2, b_oa, b_od = (
        x[:, None, :] for x in
        (b_ea, b_eo, b_e1, g_e, be_e, b_e2, b_n1, g_n, be_n, b_n2, b_oa, b_od))

    def eb(*blk):
        return pl.BlockSpec(blk, lambda e, b: (e, b) + (0,) * (len(blk) - 2))

    def ebo(*blk):
        return pl.BlockSpec(blk, lambda e, b: (e, 0, b, 0))

    def ew(*blk):
        return pl.BlockSpec(blk, lambda e, b: (e,) + (0,) * (len(blk) - 1))

    in_specs = [
        eb(1, HB, 2 * AG),
        ebo(1, NOBJ, HB, 2 * (DYN + STAT)),
        eb(1, HB, 2 * ACT),
        ew(1, AG, EMB), ew(1, 1, EMB),
        ew(1, DYN + STAT, EMB), ew(1, 1, EMB),
        ew(1, 2 * EMB + ACT, HID), ew(1, 1, HID),
        ew(1, 1, HID), ew(1, 1, HID),
        ew(1, HID, HID), ew(1, 1, HID),
        ew(1, EMB + HID + ACT, HID), ew(1, 1, HID),
        ew(1, 1, HID), ew(1, 1, HID),
        ew(1, HID, EMB), ew(1, 1, EMB),
        ew(1, EMB, AG), ew(1, 1, AG),
        ew(1, EMB, DYN), ew(1, 1, DYN),
    ]
    out_specs = (
        eb(1, HB, 2 * AG),
        ebo(1, NOBJ, HB, 2 * DYN),
    )
    out_shapes = (
        jax.ShapeDtypeStruct((NE, B // 2, 2 * AG), jnp.float32),
        jax.ShapeDtypeStruct((NE, NOBJ, B // 2, 2 * DYN), jnp.float32),
    )
    a_out, o_out = pl.pallas_call(
        _gnn_kernel,
        grid=grid,
        in_specs=in_specs,
        out_specs=out_specs,
        out_shape=out_shapes,
        compiler_params=pltpu.CompilerParams(
            dimension_semantics=("parallel", "parallel"),
        ),
    )(ag_p, obj_p, act_p,
      W_ea, b_ea, W_eo, b_eo,
      W_e1, b_e1, g_e, be_e, W_e2, b_e2,
      W_n1, b_n1, g_n, be_n, W_n2, b_n2,
      W_oa, b_oa, W_od, b_od)

    agent_out = a_out.reshape(NE, B, AG)
    obj_out = o_out.reshape(NE, NOBJ, B, DYN).transpose(0, 2, 1, 3)
    return (agent_out, obj_out)
